# baseline (device time: 62088 ns/iter reference)
import jax
import jax.numpy as jnp
from jax import lax
from jax.experimental import pallas as pl
from jax.experimental.pallas import tpu as pltpu

N_DEV = 16
B, SQ, D = 4, 256, 1024
HQ_LOC, DH = 8, 128
KV_LOC = 2
ROWS = B * SQ
PIECE = SQ // N_DEV
SCALE = 0.08838834764831843

_MESH = pl.DeviceIdType.MESH


def kernel(x, Wq, Wo, Wk, Wv):
    def body(x_ref, wq_ref, wo_ref, wk_hbm, wv_hbm, out_ref,
             wk_s, wv_s, rs_send, rs_stage, ag_src,
             local_sems, own_sems, rs_ssem, rs_rsems, ag_ssem, ag_rsem):
        d = lax.axis_index("i")

        cp_k = pltpu.make_async_copy(
            wk_hbm.at[:, pl.ds(d * KV_LOC * DH, KV_LOC * DH)],
            wk_s, local_sems.at[0])
        cp_v = pltpu.make_async_copy(
            wv_hbm.at[:, pl.ds(d * KV_LOC * DH, KV_LOC * DH)],
            wv_s, local_sems.at[1])
        cp_k.start()
        cp_v.start()

        rs_stage[pl.ds(d * B, B)] = jnp.zeros((B, PIECE, D), jnp.bfloat16)

        barrier = pltpu.get_barrier_semaphore()
        for k in range(1, N_DEV):
            pl.semaphore_signal(barrier, inc=1, device_id=((d + k) % N_DEV,),
                                device_id_type=_MESH)
        pl.semaphore_wait(barrier, N_DEV - 1)

        wq_b = wq_ref[...].astype(jnp.bfloat16)
        wo_b = wo_ref[...].astype(jnp.bfloat16)
        cp_k.wait()
        cp_v.wait()
        wk_b = wk_s[...].astype(jnp.bfloat16)
        wv_b = wv_s[...].astype(jnp.bfloat16)

        rs_desc, ag_desc, own_desc = [], [], []

        def piece_wait_recv(sem):
            dummy = rs_send.at[pl.ds(0, PIECE), :]
            pltpu.make_async_remote_copy(
                src_ref=dummy, dst_ref=dummy, send_sem=rs_ssem,
                recv_sem=sem, device_id=(d,), device_id_type=_MESH,
            ).wait_recv()

        def finalize(b):
            for _ in range(N_DEV - 1):
                piece_wait_recv(rs_rsems.at[b])
            own = rs_send[pl.ds(b * SQ + d * PIECE, PIECE), :]
            acc = own.astype(jnp.float32)
            for k in range(N_DEV):
                acc = acc + rs_stage[k * B + b].astype(jnp.float32)
            ag_src[pl.ds(b * PIECE, PIECE), :] = acc.astype(jnp.bfloat16)
            src = ag_src.at[pl.ds(b * PIECE, PIECE), :]
            dst = out_ref.at[pl.ds(b * SQ + d * PIECE, PIECE), :]
            own_cp = pltpu.make_async_copy(src, dst, own_sems.at[b])
            own_cp.start()
            own_desc.append(own_cp)
            for k in range(1, N_DEV):
                r = pltpu.make_async_remote_copy(
                    src_ref=src, dst_ref=dst,
                    send_sem=ag_ssem, recv_sem=ag_rsem,
                    device_id=((d + k) % N_DEV,), device_id_type=_MESH)
                r.start()
                ag_desc.append(r)

        for b in range(B):
            r0 = b * SQ
            xb = x_ref[pl.ds(r0, SQ), :].astype(jnp.bfloat16)
            q_b = (jnp.dot(xb, wq_b, preferred_element_type=jnp.float32)
                   * SCALE).astype(jnp.bfloat16)
            k_b = jnp.dot(xb, wk_b,
                          preferred_element_type=jnp.float32
                          ).astype(jnp.bfloat16)
            v_b = jnp.dot(xb, wv_b,
                          preferred_element_type=jnp.float32
                          ).astype(jnp.bfloat16)
            outs = []
            for h in range(HQ_LOC):
                g = h // 4
                q = q_b[:, h * DH:(h + 1) * DH]
                kk = k_b[:, g * DH:(g + 1) * DH]
                vv = v_b[:, g * DH:(g + 1) * DH]
                s = lax.dot_general(q, kk, (((1,), (1,)), ((), ())),
                                    preferred_element_type=jnp.float32)
                m = jnp.max(s, axis=1, keepdims=True)
                p = jnp.exp(s - m)
                den = jnp.sum(p, axis=1, keepdims=True)
                pn = (p / den).astype(jnp.bfloat16)
                outs.append(jnp.dot(pn, vv,
                                    preferred_element_type=jnp.float32))
            attn_b = jnp.concatenate(outs, axis=1).astype(jnp.bfloat16)
            pc = jnp.dot(attn_b, wo_b, preferred_element_type=jnp.float32)
            rs_send[pl.ds(r0, SQ), :] = pc.astype(jnp.bfloat16)

            for k in range(1, N_DEV):
                dest = (d + k) % N_DEV
                r = pltpu.make_async_remote_copy(
                    src_ref=rs_send.at[pl.ds(r0 + dest * PIECE, PIECE), :],
                    dst_ref=rs_stage.at[d * B + b],
                    send_sem=rs_ssem, recv_sem=rs_rsems.at[b],
                    device_id=(dest,), device_id_type=_MESH)
                r.start()
                rs_desc.append(r)

            if b >= 1:
                finalize(b - 1)
        finalize(B - 1)

        for _ in range(B * (N_DEV - 1)):
            piece_wait_recv(ag_rsem)

        for r in own_desc:
            r.wait()
        for r in rs_desc:
            r.wait_send()
        for r in ag_desc:
            r.wait_send()

    out = pl.pallas_call(
        body,
        out_shape=jax.ShapeDtypeStruct((ROWS, D), jnp.bfloat16),
        in_specs=[
            pl.BlockSpec(memory_space=pltpu.VMEM),
            pl.BlockSpec(memory_space=pltpu.VMEM),
            pl.BlockSpec(memory_space=pltpu.VMEM),
            pl.BlockSpec(memory_space=pltpu.HBM),
            pl.BlockSpec(memory_space=pltpu.HBM),
        ],
        out_specs=pl.BlockSpec(memory_space=pltpu.HBM),
        scratch_shapes=[
            pltpu.VMEM((D, KV_LOC * DH), jnp.float32),
            pltpu.VMEM((D, KV_LOC * DH), jnp.float32),
            pltpu.VMEM((ROWS, D), jnp.bfloat16),
            pltpu.VMEM((N_DEV * B, PIECE, D), jnp.bfloat16),
            pltpu.VMEM((B * PIECE, D), jnp.bfloat16),
            pltpu.SemaphoreType.DMA((2,)),
            pltpu.SemaphoreType.DMA((B,)),
            pltpu.SemaphoreType.DMA,
            pltpu.SemaphoreType.DMA((B,)),
            pltpu.SemaphoreType.DMA,
            pltpu.SemaphoreType.DMA,
        ],
        compiler_params=pltpu.CompilerParams(collective_id=0),
    )(x.reshape(ROWS, D), Wq, Wo, Wk, Wv)
    return out.reshape(B, SQ, D)
